# Initial kernel scaffold; baseline (speedup 1.0000x reference)
import jax
import jax.numpy as jnp
from jax.experimental import pallas as pl


def kernel(text, ctx, token_embedding):
    raise NotImplementedError("write your pallas kernel here")


# speedup vs baseline: 1.1636x; 1.1636x over previous
import jax
import jax.numpy as jnp
from jax.experimental import pallas as pl
from jax.experimental.pallas import tpu as pltpu
from jax.experimental.pallas import tpu_sc as plsc

B = 1024
SEQ = 77
CTX_DIM = 512
N_CTX = 4
KEEP = SEQ - N_CTX
IDX_PAD = 80
SL, LN = 4, 128

NUM_CORES = 2
NUM_SUBCORES = 16
NUM_WORKERS = NUM_CORES * NUM_SUBCORES
BATCH_PER_W = B // NUM_WORKERS
NBUF = 2


def _sc_body(idx_hbm, ctx_hbm, table_hbm, out_hbm,
             idx_v, ctx_v, rows0, rows1, gsem0, gsem1, wsem0, wsem1):
    wid = jax.lax.axis_index("s") * NUM_CORES + jax.lax.axis_index("c")
    base = wid * BATCH_PER_W

    rows = (rows0, rows1)
    gsems = (gsem0, gsem1)
    wsems = (wsem0, wsem1)

    pltpu.sync_copy(idx_hbm.at[pl.ds(base, BATCH_PER_W)], idx_v)
    pltpu.sync_copy(ctx_hbm, ctx_v)

    def gather_copy(j, b):
        return pltpu.make_async_copy(
            table_hbm.at[idx_v.at[j, pl.ds(0, KEEP)]],
            rows[b],
            gsems[b])

    def write_copy(j, b):
        return pltpu.make_async_copy(
            rows[b],
            out_hbm.at[base + j].at[pl.ds(N_CTX, KEEP)],
            wsems[b])

    gather_copy(0, 0).start()
    gather_copy(1, 1).start()

    def step(j, b, refire):
        pltpu.sync_copy(ctx_v, out_hbm.at[base + j].at[pl.ds(0, N_CTX)])
        gather_copy(j, b).wait()
        write_copy(j, b).start()
        write_copy(j, b).wait()
        if refire:
            gather_copy(j + NBUF, b).start()

    @pl.loop(0, BATCH_PER_W // NBUF - 1)
    def _(g):
        for b in range(NBUF):
            step(NBUF * g + b, b, refire=True)

    for b in range(NBUF):
        step(BATCH_PER_W - NBUF + b, b, refire=False)


@jax.jit
def _prompt_gather(idx, ctx, table):
    grid_kernel = pl.kernel(
        _sc_body,
        out_type=jax.ShapeDtypeStruct((B, SEQ, SL, LN), jnp.float32),
        mesh=plsc.VectorSubcoreMesh(core_axis_name="c", subcore_axis_name="s"),
        scratch_types=[
            pltpu.VMEM((BATCH_PER_W, IDX_PAD), jnp.int32),
            pltpu.VMEM((N_CTX, SL, LN), jnp.float32),
            pltpu.VMEM((KEEP, SL, LN), jnp.float32),
            pltpu.VMEM((KEEP, SL, LN), jnp.float32),
            pltpu.SemaphoreType.DMA,
            pltpu.SemaphoreType.DMA,
            pltpu.SemaphoreType.DMA,
            pltpu.SemaphoreType.DMA,
        ],
    )
    return grid_kernel(idx, ctx, table)


def kernel(text, ctx, token_embedding):
    idx = jnp.pad(text[:, :KEEP], ((0, 0), (0, IDX_PAD - KEEP)))
    ctx4 = ctx.reshape(N_CTX, SL, LN)
    table4 = token_embedding.reshape(-1, SL, LN)
    out = _prompt_gather(idx, ctx4, table4)
    return out.reshape(B, SEQ, CTX_DIM)
